# baseline (speedup 1.0000x reference)
import functools

import jax
import jax.numpy as jnp
from jax import lax
from jax.experimental import pallas as pl
from jax.experimental.pallas import tpu as pltpu
from jax.experimental.pallas import tpu_sc as plsc

_NW = 32
_CH = 16384
_NBUF = 2
_LANES = 16


def _sc_body(w_hbm, x_hbm, o_hbm, wv, in0, in1, out0, out1, in_sem, out_sem):
    c = lax.axis_index("c")
    s = lax.axis_index("s")
    wid = s * 2 + c
    per_w = x_hbm.shape[0] // _NW
    base = wid * per_w
    n_chunks = per_w // _CH

    pltpu.sync_copy(w_hbm, wv)
    f2 = wv[...] * wv[...]

    in_bufs = (in0, in1)
    out_bufs = (out0, out1)

    def in_copy(g, b):
        return pltpu.make_async_copy(
            x_hbm.at[pl.ds(base + g * _CH, _CH)], in_bufs[b], in_sem.at[b])

    def out_copy(g, b):
        return pltpu.make_async_copy(
            out_bufs[b], o_hbm.at[pl.ds(base + g * _CH, _CH)], out_sem.at[b])

    for b in range(_NBUF):
        in_copy(b, b).start()

    def outer(g2, carry):
        for b in range(_NBUF):
            g = g2 * _NBUF + b
            in_copy(g, b).wait()

            @pl.when(g2 > 0)
            def _wait_out():
                out_copy(g - _NBUF, b).wait()

            def compute(j, carry2):
                off = j * (8 * _LANES)
                for k in range(8):
                    sl = pl.ds(off + k * _LANES, _LANES)
                    out_bufs[b][sl] = in_bufs[b][sl] * f2
                return carry2

            lax.fori_loop(0, _CH // (8 * _LANES), compute, 0)
            out_copy(g, b).start()

            @pl.when(g + _NBUF < n_chunks)
            def _next_in():
                in_copy(g + _NBUF, b).start()

        return carry

    lax.fori_loop(0, n_chunks // _NBUF, outer, 0)

    for b in range(_NBUF):
        out_copy(n_chunks - _NBUF + b, b).wait()


def kernel(x, multiple_factor_weight):
    b, r, c = x.shape
    n = b * r * c
    x_flat = x.reshape(n)
    w16 = jnp.broadcast_to(multiple_factor_weight.reshape(1), (_LANES,))
    sck = functools.partial(
        pl.kernel,
        out_type=jax.ShapeDtypeStruct((n,), x.dtype),
        mesh=plsc.VectorSubcoreMesh(core_axis_name="c", subcore_axis_name="s"),
        scratch_types=[
            pltpu.VMEM((_LANES,), jnp.float32),
            pltpu.VMEM((_CH,), jnp.float32),
            pltpu.VMEM((_CH,), jnp.float32),
            pltpu.VMEM((_CH,), jnp.float32),
            pltpu.VMEM((_CH,), jnp.float32),
            pltpu.SemaphoreType.DMA((_NBUF,)),
            pltpu.SemaphoreType.DMA((_NBUF,)),
        ],
    )(_sc_body)
    out = sck(w16, x_flat)
    return out.reshape(b, r, c)
